# Initial kernel scaffold; baseline (speedup 1.0000x reference)
import functools

import jax
import jax.numpy as jnp
from jax import lax
from jax.experimental import pallas as pl
from jax.experimental.pallas import tpu as pltpu

N_NODES = 10000
N_GRAPHS = 128
HIDDEN = 512
N_LAYERS = 4
EMB = HIDDEN * N_LAYERS

_BLK = 500
_NBLK = N_NODES // _BLK
_PREC = lax.Precision.HIGHEST


def _mlp_body(h_ref, agg_ref, w1_ref, b1_ref, w2_ref, b2_ref,
              z_ref, stats_ref, acc_ref):
    t = h_ref[...] + agg_ref[...]
    z1 = jnp.maximum(
        lax.dot_general(t, w1_ref[...], (((1,), (0,)), ((), ())),
                        precision=_PREC, preferred_element_type=jnp.float32)
        + b1_ref[...], 0.0)
    z2 = lax.dot_general(z1, w2_ref[...], (((1,), (0,)), ((), ())),
                         precision=_PREC, preferred_element_type=jnp.float32) \
        + b2_ref[...]
    z2 = jnp.maximum(z2, 0.0)
    z_ref[...] = z2

    @pl.when(pl.program_id(0) == 0)
    def _():
        acc_ref[...] = jnp.zeros_like(acc_ref)

    acc_ref[0, :] += jnp.sum(z2, axis=0)
    acc_ref[1, :] += jnp.sum(z2 * z2, axis=0)

    @pl.when(pl.program_id(0) == _NBLK - 1)
    def _():
        stats_ref[...] = acc_ref[...]


def _mlp_layer(h, agg, w1, b1, w2, b2):
    d_in = h.shape[1]
    z, stats = pl.pallas_call(
        _mlp_body,
        grid=(_NBLK,),
        in_specs=[
            pl.BlockSpec((_BLK, d_in), lambda i: (i, 0)),
            pl.BlockSpec((_BLK, d_in), lambda i: (i, 0)),
            pl.BlockSpec((d_in, HIDDEN), lambda i: (0, 0)),
            pl.BlockSpec((1, HIDDEN), lambda i: (0, 0)),
            pl.BlockSpec((HIDDEN, HIDDEN), lambda i: (0, 0)),
            pl.BlockSpec((1, HIDDEN), lambda i: (0, 0)),
        ],
        out_specs=[
            pl.BlockSpec((_BLK, HIDDEN), lambda i: (i, 0)),
            pl.BlockSpec((2, HIDDEN), lambda i: (0, 0)),
        ],
        out_shape=[
            jax.ShapeDtypeStruct((N_NODES, HIDDEN), jnp.float32),
            jax.ShapeDtypeStruct((2, HIDDEN), jnp.float32),
        ],
        scratch_shapes=[pltpu.VMEM((2, HIDDEN), jnp.float32)],
    )(h, agg, w1, b1.reshape(1, HIDDEN), w2, b2.reshape(1, HIDDEN))
    return z, stats


def _bn_body(z_ref, stats_ref, g_ref, b_ref, out_ref):
    mean = stats_ref[0, :] / N_NODES
    var = stats_ref[1, :] / N_NODES - mean * mean
    rstd = lax.rsqrt(var + 1e-5)
    out_ref[...] = (z_ref[...] - mean[None, :]) * (rstd * g_ref[0, :])[None, :] \
        + b_ref[0, :][None, :]


def _bn_layer(z, stats, g, b):
    return pl.pallas_call(
        _bn_body,
        grid=(_NBLK,),
        in_specs=[
            pl.BlockSpec((_BLK, HIDDEN), lambda i: (i, 0)),
            pl.BlockSpec((2, HIDDEN), lambda i: (0, 0)),
            pl.BlockSpec((1, HIDDEN), lambda i: (0, 0)),
            pl.BlockSpec((1, HIDDEN), lambda i: (0, 0)),
        ],
        out_specs=pl.BlockSpec((_BLK, HIDDEN), lambda i: (i, 0)),
        out_shape=jax.ShapeDtypeStruct((N_NODES, HIDDEN), jnp.float32),
    )(z, stats, g.reshape(1, HIDDEN), b.reshape(1, HIDDEN))


def _pool_proj_body(batch_ref, m_ref, wp1_ref, bp1_ref, wp2_ref, bp2_ref,
                    out_ref, pool_ref):
    @pl.when(pl.program_id(0) == 0)
    def _():
        pool_ref[...] = jnp.zeros_like(pool_ref)

    gids = lax.broadcasted_iota(jnp.int32, (N_GRAPHS, _BLK), 0)
    onehot = (batch_ref[0, 0, :][None, :] == gids).astype(jnp.float32)
    pool_ref[...] += lax.dot_general(
        onehot, m_ref[...], (((1,), (0,)), ((), ())),
        precision=_PREC, preferred_element_type=jnp.float32)

    @pl.when(pl.program_id(0) == _NBLK - 1)
    def _():
        y = pool_ref[...]
        h1 = jnp.maximum(
            lax.dot_general(y, wp1_ref[...], (((1,), (0,)), ((), ())),
                            precision=_PREC,
                            preferred_element_type=jnp.float32)
            + bp1_ref[...], 0.0)
        out_ref[...] = lax.dot_general(
            h1, wp2_ref[...], (((1,), (0,)), ((), ())),
            precision=_PREC, preferred_element_type=jnp.float32) + bp2_ref[...]


def _pool_proj(batch, m, wp1, bp1, wp2, bp2):
    batch3 = batch.reshape(_NBLK, 1, _BLK)
    return pl.pallas_call(
        _pool_proj_body,
        grid=(_NBLK,),
        in_specs=[
            pl.BlockSpec((1, 1, _BLK), lambda i: (i, 0, 0)),
            pl.BlockSpec((_BLK, EMB), lambda i: (i, 0)),
            pl.BlockSpec((EMB, EMB), lambda i: (0, 0)),
            pl.BlockSpec((1, EMB), lambda i: (0, 0)),
            pl.BlockSpec((EMB, EMB), lambda i: (0, 0)),
            pl.BlockSpec((1, EMB), lambda i: (0, 0)),
        ],
        out_specs=pl.BlockSpec((N_GRAPHS, EMB), lambda i: (0, 0)),
        out_shape=jax.ShapeDtypeStruct((N_GRAPHS, EMB), jnp.float32),
        scratch_shapes=[pltpu.VMEM((N_GRAPHS, EMB), jnp.float32)],
    )(batch3, m, wp1, bp1.reshape(1, EMB), wp2, bp2.reshape(1, EMB))


def _aggregate(h, edge_index):
    return jax.ops.segment_sum(h[edge_index[0]], edge_index[1],
                               num_segments=N_NODES)


def kernel(x, edge_index, batch, params):
    h = x
    xs = []
    for l in range(N_LAYERS):
        agg = _aggregate(h, edge_index)
        z, stats = _mlp_layer(h, agg, params[f"W1_{l}"], params[f"b1_{l}"],
                              params[f"W2_{l}"], params[f"b2_{l}"])
        z = _bn_layer(z, stats, params[f"bn_g_{l}"], params[f"bn_b_{l}"])
        xs.append(z)
        h = z
    m = jnp.concatenate(xs, axis=1)
    return _pool_proj(batch, m, params["Wp1"], params["bp1"],
                      params["Wp2"], params["bp2"])


# speedup vs baseline: 1.0332x; 1.0332x over previous
import functools

import jax
import jax.numpy as jnp
from jax import lax
from jax.experimental import pallas as pl
from jax.experimental.pallas import tpu as pltpu

N_NODES = 10000
N_GRAPHS = 128
HIDDEN = 512
N_LAYERS = 4
EMB = HIDDEN * N_LAYERS

_BLK = 1000
_NBLK = N_NODES // _BLK
_PREC = lax.Precision.DEFAULT
_PREC_HI = lax.Precision.HIGHEST


def _mlp_body(h_ref, agg_ref, w1_ref, b1_ref, w2_ref, b2_ref,
              z_ref, stats_ref, acc_ref):
    t = h_ref[...] + agg_ref[...]
    z1 = jnp.maximum(
        lax.dot_general(t, w1_ref[...], (((1,), (0,)), ((), ())),
                        precision=_PREC, preferred_element_type=jnp.float32)
        + b1_ref[...], 0.0)
    z2 = lax.dot_general(z1, w2_ref[...], (((1,), (0,)), ((), ())),
                         precision=_PREC, preferred_element_type=jnp.float32) \
        + b2_ref[...]
    z2 = jnp.maximum(z2, 0.0)
    z_ref[...] = z2

    @pl.when(pl.program_id(0) == 0)
    def _():
        acc_ref[...] = jnp.zeros_like(acc_ref)

    acc_ref[0, :] += jnp.sum(z2, axis=0)
    acc_ref[1, :] += jnp.sum(z2 * z2, axis=0)

    @pl.when(pl.program_id(0) == _NBLK - 1)
    def _():
        stats_ref[...] = acc_ref[...]


def _mlp_layer(h, agg, w1, b1, w2, b2):
    d_in = h.shape[1]
    z, stats = pl.pallas_call(
        _mlp_body,
        grid=(_NBLK,),
        in_specs=[
            pl.BlockSpec((_BLK, d_in), lambda i: (i, 0)),
            pl.BlockSpec((_BLK, d_in), lambda i: (i, 0)),
            pl.BlockSpec((d_in, HIDDEN), lambda i: (0, 0)),
            pl.BlockSpec((1, HIDDEN), lambda i: (0, 0)),
            pl.BlockSpec((HIDDEN, HIDDEN), lambda i: (0, 0)),
            pl.BlockSpec((1, HIDDEN), lambda i: (0, 0)),
        ],
        out_specs=[
            pl.BlockSpec((_BLK, HIDDEN), lambda i: (i, 0)),
            pl.BlockSpec((2, HIDDEN), lambda i: (0, 0)),
        ],
        out_shape=[
            jax.ShapeDtypeStruct((N_NODES, HIDDEN), jnp.float32),
            jax.ShapeDtypeStruct((2, HIDDEN), jnp.float32),
        ],
        scratch_shapes=[pltpu.VMEM((2, HIDDEN), jnp.float32)],
    )(h, agg, w1, b1.reshape(1, HIDDEN), w2, b2.reshape(1, HIDDEN))
    return z, stats


def _bn_body(z_ref, stats_ref, g_ref, b_ref, out_ref):
    mean = stats_ref[0, :] / N_NODES
    var = stats_ref[1, :] / N_NODES - mean * mean
    rstd = lax.rsqrt(var + 1e-5)
    out_ref[...] = (z_ref[...] - mean[None, :]) * (rstd * g_ref[0, :])[None, :] \
        + b_ref[0, :][None, :]


def _bn_layer(z, stats, g, b):
    return pl.pallas_call(
        _bn_body,
        grid=(_NBLK,),
        in_specs=[
            pl.BlockSpec((_BLK, HIDDEN), lambda i: (i, 0)),
            pl.BlockSpec((2, HIDDEN), lambda i: (0, 0)),
            pl.BlockSpec((1, HIDDEN), lambda i: (0, 0)),
            pl.BlockSpec((1, HIDDEN), lambda i: (0, 0)),
        ],
        out_specs=pl.BlockSpec((_BLK, HIDDEN), lambda i: (i, 0)),
        out_shape=jax.ShapeDtypeStruct((N_NODES, HIDDEN), jnp.float32),
    )(z, stats, g.reshape(1, HIDDEN), b.reshape(1, HIDDEN))


def _pool_proj_body(batch_ref, m_ref, wp1_ref, bp1_ref, wp2_ref, bp2_ref,
                    out_ref, pool_ref):
    @pl.when(pl.program_id(0) == 0)
    def _():
        pool_ref[...] = jnp.zeros_like(pool_ref)

    gids = lax.broadcasted_iota(jnp.int32, (N_GRAPHS, _BLK), 0)
    onehot = (batch_ref[0, 0, :][None, :] == gids).astype(jnp.float32)
    pool_ref[...] += lax.dot_general(
        onehot, m_ref[...], (((1,), (0,)), ((), ())),
        precision=_PREC_HI, preferred_element_type=jnp.float32)

    @pl.when(pl.program_id(0) == _NBLK - 1)
    def _():
        y = pool_ref[...]
        h1 = jnp.maximum(
            lax.dot_general(y, wp1_ref[...], (((1,), (0,)), ((), ())),
                            precision=_PREC,
                            preferred_element_type=jnp.float32)
            + bp1_ref[...], 0.0)
        out_ref[...] = lax.dot_general(
            h1, wp2_ref[...], (((1,), (0,)), ((), ())),
            precision=_PREC, preferred_element_type=jnp.float32) + bp2_ref[...]


def _pool_proj(batch, m, wp1, bp1, wp2, bp2):
    batch3 = batch.reshape(_NBLK, 1, _BLK)
    return pl.pallas_call(
        _pool_proj_body,
        grid=(_NBLK,),
        in_specs=[
            pl.BlockSpec((1, 1, _BLK), lambda i: (i, 0, 0)),
            pl.BlockSpec((_BLK, EMB), lambda i: (i, 0)),
            pl.BlockSpec((EMB, EMB), lambda i: (0, 0)),
            pl.BlockSpec((1, EMB), lambda i: (0, 0)),
            pl.BlockSpec((EMB, EMB), lambda i: (0, 0)),
            pl.BlockSpec((1, EMB), lambda i: (0, 0)),
        ],
        out_specs=pl.BlockSpec((N_GRAPHS, EMB), lambda i: (0, 0)),
        out_shape=jax.ShapeDtypeStruct((N_GRAPHS, EMB), jnp.float32),
        scratch_shapes=[pltpu.VMEM((N_GRAPHS, EMB), jnp.float32)],
    )(batch3, m, wp1, bp1.reshape(1, EMB), wp2, bp2.reshape(1, EMB))


def _aggregate(h, edge_index):
    return jax.ops.segment_sum(h[edge_index[0]], edge_index[1],
                               num_segments=N_NODES)


def kernel(x, edge_index, batch, params):
    h = x
    xs = []
    for l in range(N_LAYERS):
        agg = _aggregate(h, edge_index)
        z, stats = _mlp_layer(h, agg, params[f"W1_{l}"], params[f"b1_{l}"],
                              params[f"W2_{l}"], params[f"b2_{l}"])
        z = _bn_layer(z, stats, params[f"bn_g_{l}"], params[f"bn_b_{l}"])
        xs.append(z)
        h = z
    m = jnp.concatenate(xs, axis=1)
    return _pool_proj(batch, m, params["Wp1"], params["bp1"],
                      params["Wp2"], params["bp2"])


# speedup vs baseline: 1.2533x; 1.2130x over previous
import functools

import jax
import jax.numpy as jnp
from jax import lax
from jax.experimental import pallas as pl
from jax.experimental.pallas import tpu as pltpu

N_NODES = 10000
N_GRAPHS = 128
HIDDEN = 512
N_LAYERS = 4
EMB = HIDDEN * N_LAYERS

_BLK = 1000
_NBLK = N_NODES // _BLK
_PREC = lax.Precision.DEFAULT
_PREC_HI = lax.Precision.HIGHEST


def _mlp_body(h_ref, agg_ref, w1_ref, b1_ref, w2_ref, b2_ref,
              z_ref, stats_ref, acc_ref):
    t = h_ref[...] + agg_ref[...]
    z1 = jnp.maximum(
        lax.dot_general(t, w1_ref[...], (((1,), (0,)), ((), ())),
                        precision=_PREC, preferred_element_type=jnp.float32)
        + b1_ref[...], 0.0)
    z2 = lax.dot_general(z1, w2_ref[...], (((1,), (0,)), ((), ())),
                         precision=_PREC, preferred_element_type=jnp.float32) \
        + b2_ref[...]
    z2 = jnp.maximum(z2, 0.0)
    z_ref[...] = z2

    @pl.when(pl.program_id(0) == 0)
    def _():
        acc_ref[...] = jnp.zeros_like(acc_ref)

    acc_ref[0, :] += jnp.sum(z2, axis=0)
    acc_ref[1, :] += jnp.sum(z2 * z2, axis=0)

    @pl.when(pl.program_id(0) == _NBLK - 1)
    def _():
        stats_ref[...] = acc_ref[...]


def _mlp_layer(h, agg, w1, b1, w2, b2):
    d_in = h.shape[1]
    z, stats = pl.pallas_call(
        _mlp_body,
        grid=(_NBLK,),
        in_specs=[
            pl.BlockSpec((_BLK, d_in), lambda i: (i, 0)),
            pl.BlockSpec((_BLK, d_in), lambda i: (i, 0)),
            pl.BlockSpec((d_in, HIDDEN), lambda i: (0, 0)),
            pl.BlockSpec((1, HIDDEN), lambda i: (0, 0)),
            pl.BlockSpec((HIDDEN, HIDDEN), lambda i: (0, 0)),
            pl.BlockSpec((1, HIDDEN), lambda i: (0, 0)),
        ],
        out_specs=[
            pl.BlockSpec((_BLK, HIDDEN), lambda i: (i, 0)),
            pl.BlockSpec((2, HIDDEN), lambda i: (0, 0)),
        ],
        out_shape=[
            jax.ShapeDtypeStruct((N_NODES, HIDDEN), jnp.float32),
            jax.ShapeDtypeStruct((2, HIDDEN), jnp.float32),
        ],
        scratch_shapes=[pltpu.VMEM((2, HIDDEN), jnp.float32)],
    )(h, agg, w1, b1.reshape(1, HIDDEN), w2, b2.reshape(1, HIDDEN))
    return z, stats


def _bn_body(z_ref, stats_ref, g_ref, b_ref, out_ref):
    mean = stats_ref[0, :] / N_NODES
    var = stats_ref[1, :] / N_NODES - mean * mean
    rstd = lax.rsqrt(var + 1e-5)
    out_ref[...] = (z_ref[...] - mean[None, :]) * (rstd * g_ref[0, :])[None, :] \
        + b_ref[0, :][None, :]


def _bn_layer(z, stats, g, b):
    return pl.pallas_call(
        _bn_body,
        grid=(_NBLK,),
        in_specs=[
            pl.BlockSpec((_BLK, HIDDEN), lambda i: (i, 0)),
            pl.BlockSpec((2, HIDDEN), lambda i: (0, 0)),
            pl.BlockSpec((1, HIDDEN), lambda i: (0, 0)),
            pl.BlockSpec((1, HIDDEN), lambda i: (0, 0)),
        ],
        out_specs=pl.BlockSpec((_BLK, HIDDEN), lambda i: (i, 0)),
        out_shape=jax.ShapeDtypeStruct((N_NODES, HIDDEN), jnp.float32),
    )(z, stats, g.reshape(1, HIDDEN), b.reshape(1, HIDDEN))


def _pool_proj_body(batch_ref, m_ref, wp1_ref, bp1_ref, wp2_ref, bp2_ref,
                    out_ref, pool_ref):
    @pl.when(pl.program_id(0) == 0)
    def _():
        pool_ref[...] = jnp.zeros_like(pool_ref)

    gids = lax.broadcasted_iota(jnp.int32, (N_GRAPHS, _BLK), 0)
    onehot = (batch_ref[0, 0, :][None, :] == gids).astype(jnp.float32)
    pool_ref[...] += lax.dot_general(
        onehot, m_ref[...], (((1,), (0,)), ((), ())),
        precision=_PREC_HI, preferred_element_type=jnp.float32)

    @pl.when(pl.program_id(0) == _NBLK - 1)
    def _():
        y = pool_ref[...]
        h1 = jnp.maximum(
            lax.dot_general(y, wp1_ref[...], (((1,), (0,)), ((), ())),
                            precision=_PREC,
                            preferred_element_type=jnp.float32)
            + bp1_ref[...], 0.0)
        out_ref[...] = lax.dot_general(
            h1, wp2_ref[...], (((1,), (0,)), ((), ())),
            precision=_PREC, preferred_element_type=jnp.float32) + bp2_ref[...]


def _pool_proj(batch, m, wp1, bp1, wp2, bp2):
    batch3 = batch.reshape(_NBLK, 1, _BLK)
    return pl.pallas_call(
        _pool_proj_body,
        grid=(_NBLK,),
        in_specs=[
            pl.BlockSpec((1, 1, _BLK), lambda i: (i, 0, 0)),
            pl.BlockSpec((_BLK, EMB), lambda i: (i, 0)),
            pl.BlockSpec((EMB, EMB), lambda i: (0, 0)),
            pl.BlockSpec((1, EMB), lambda i: (0, 0)),
            pl.BlockSpec((EMB, EMB), lambda i: (0, 0)),
            pl.BlockSpec((1, EMB), lambda i: (0, 0)),
        ],
        out_specs=pl.BlockSpec((N_GRAPHS, EMB), lambda i: (0, 0)),
        out_shape=jax.ShapeDtypeStruct((N_GRAPHS, EMB), jnp.float32),
        scratch_shapes=[pltpu.VMEM((N_GRAPHS, EMB), jnp.float32)],
    )(batch3, m, wp1, bp1.reshape(1, EMB), wp2, bp2.reshape(1, EMB))


from jax.experimental.pallas import tpu_sc as plsc

N_EDGES = 160000
_CHUNK = 64
_NCHUNK = N_EDGES // _CHUNK
_RNG = 160
_NRANGE = 64
_NPAD = _RNG * _NRANGE


def _make_sc_agg(d_feat):
    mesh = plsc.VectorSubcoreMesh(core_axis_name="c", subcore_axis_name="s")
    nv = d_feat // 16

    @functools.partial(
        pl.kernel,
        out_type=jax.ShapeDtypeStruct((_NPAD, d_feat), jnp.float32),
        mesh=mesh,
        scratch_types=[
            pltpu.VMEM((_CHUNK,), jnp.int32),
            pltpu.VMEM((_CHUNK,), jnp.int32),
            pltpu.VMEM((_CHUNK, d_feat), jnp.float32),
            pltpu.VMEM((_RNG, d_feat), jnp.float32),
            pltpu.VMEM((16,), jnp.int32),
            pltpu.SemaphoreType.DMA,
        ],
    )
    def sc_agg(h_hbm, srcs_hbm, dsts_hbm, offs_hbm, out_hbm,
               src_i, dst_i, rows, acc, ofs16, sem):
        c = lax.axis_index("c")
        s = lax.axis_index("s")
        wid = c * 16 + s

        pltpu.sync_copy(offs_hbm.at[pl.ds(wid * 16, 16)], ofs16)
        ov = ofs16[pl.ds(0, 16)]

        @pl.loop(0, 2)
        def _pass(p):
            c0 = jnp.where(p == 0, ov[0], ov[2])
            c1 = jnp.where(p == 0, ov[1], ov[3])
            lo = (wid * 2 + p) * _RNG

            @pl.loop(0, _RNG)
            def _za(i):
                for j in range(nv):
                    acc[i, pl.ds(j * 16, 16)] = jnp.zeros((16,), jnp.float32)

            @pl.loop(c0, c1)
            def _chunk(ci):
                pltpu.sync_copy(srcs_hbm.at[pl.ds(ci * _CHUNK, _CHUNK)], src_i)
                pltpu.sync_copy(dsts_hbm.at[pl.ds(ci * _CHUNK, _CHUNK)], dst_i)
                pltpu.async_copy(h_hbm.at[src_i], rows, sem).wait()

                @pl.loop(0, _CHUNK // 16)
                def _grp(g):
                    dv = dst_i[pl.ds(g * 16, 16)]
                    for l in range(16):
                        dl = dv[l] - lo
                        k = g * 16 + l

                        @pl.when((dl >= 0) & (dl < _RNG))
                        def _():
                            for j in range(nv):
                                acc[dl, pl.ds(j * 16, 16)] += \
                                    rows[k, pl.ds(j * 16, 16)]

            pltpu.sync_copy(acc, out_hbm.at[pl.ds((wid * 2 + p) * _RNG,
                                                  _RNG), :])

    return sc_agg


_sc_agg = {256: _make_sc_agg(256), 512: _make_sc_agg(512)}


def _sort_edges(edge_index):
    order = jnp.argsort(edge_index[1])
    srcs = edge_index[0][order]
    dsts = edge_index[1][order]
    bounds = jnp.searchsorted(
        dsts, jnp.arange(0, _NPAD + 1, _RNG, dtype=jnp.int32)).astype(jnp.int32)
    cs = bounds[:-1] // _CHUNK
    ce = (bounds[1:] + _CHUNK - 1) // _CHUNK
    ofs = jnp.zeros((32, 16), jnp.int32)
    ofs = ofs.at[:, 0].set(cs[0::2]).at[:, 1].set(ce[0::2])
    ofs = ofs.at[:, 2].set(cs[1::2]).at[:, 3].set(ce[1::2])
    return srcs, dsts, ofs.reshape(512)


def _aggregate(h, srcs, dsts, offs):
    return _sc_agg[h.shape[1]](h, srcs, dsts, offs)


def kernel(x, edge_index, batch, params):
    srcs, dsts, offs = _sort_edges(edge_index)
    h = x
    xs = []
    for l in range(N_LAYERS):
        agg = _aggregate(h, srcs, dsts, offs)
        z, stats = _mlp_layer(h, agg, params[f"W1_{l}"], params[f"b1_{l}"],
                              params[f"W2_{l}"], params[f"b2_{l}"])
        z = _bn_layer(z, stats, params[f"bn_g_{l}"], params[f"bn_b_{l}"])
        xs.append(z)
        h = z
    m = jnp.concatenate(xs, axis=1)
    return _pool_proj(batch, m, params["Wp1"], params["bp1"],
                      params["Wp2"], params["bp2"])


# speedup vs baseline: 1.3223x; 1.0551x over previous
import functools

import jax
import jax.numpy as jnp
from jax import lax
from jax.experimental import pallas as pl
from jax.experimental.pallas import tpu as pltpu

N_NODES = 10000
N_GRAPHS = 128
HIDDEN = 512
N_LAYERS = 4
EMB = HIDDEN * N_LAYERS

_BLK = 1000
_NBLK = N_NODES // _BLK
_PREC = lax.Precision.DEFAULT
_PREC_HI = lax.Precision.HIGHEST


def _mlp_body(h_ref, agg_ref, w1_ref, b1_ref, w2_ref, b2_ref,
              z_ref, stats_ref, acc_ref):
    t = h_ref[...] + agg_ref[...]
    z1 = jnp.maximum(
        lax.dot_general(t, w1_ref[...], (((1,), (0,)), ((), ())),
                        precision=_PREC, preferred_element_type=jnp.float32)
        + b1_ref[...], 0.0)
    z2 = lax.dot_general(z1, w2_ref[...], (((1,), (0,)), ((), ())),
                         precision=_PREC, preferred_element_type=jnp.float32) \
        + b2_ref[...]
    z2 = jnp.maximum(z2, 0.0)
    z_ref[...] = z2

    @pl.when(pl.program_id(0) == 0)
    def _():
        acc_ref[...] = jnp.zeros_like(acc_ref)

    acc_ref[0, :] += jnp.sum(z2, axis=0)
    acc_ref[1, :] += jnp.sum(z2 * z2, axis=0)

    @pl.when(pl.program_id(0) == _NBLK - 1)
    def _():
        stats_ref[...] = acc_ref[...]


def _mlp_layer(h, agg, w1, b1, w2, b2):
    d_in = h.shape[1]
    z, stats = pl.pallas_call(
        _mlp_body,
        grid=(_NBLK,),
        in_specs=[
            pl.BlockSpec((_BLK, d_in), lambda i: (i, 0)),
            pl.BlockSpec((_BLK, d_in), lambda i: (i, 0)),
            pl.BlockSpec((d_in, HIDDEN), lambda i: (0, 0)),
            pl.BlockSpec((1, HIDDEN), lambda i: (0, 0)),
            pl.BlockSpec((HIDDEN, HIDDEN), lambda i: (0, 0)),
            pl.BlockSpec((1, HIDDEN), lambda i: (0, 0)),
        ],
        out_specs=[
            pl.BlockSpec((_BLK, HIDDEN), lambda i: (i, 0)),
            pl.BlockSpec((2, HIDDEN), lambda i: (0, 0)),
        ],
        out_shape=[
            jax.ShapeDtypeStruct((N_NODES, HIDDEN), jnp.float32),
            jax.ShapeDtypeStruct((2, HIDDEN), jnp.float32),
        ],
        scratch_shapes=[pltpu.VMEM((2, HIDDEN), jnp.float32)],
    )(h, agg, w1, b1.reshape(1, HIDDEN), w2, b2.reshape(1, HIDDEN))
    return z, stats


def _bn_body(z_ref, stats_ref, g_ref, b_ref, out_ref):
    mean = stats_ref[0, :] / N_NODES
    var = stats_ref[1, :] / N_NODES - mean * mean
    rstd = lax.rsqrt(var + 1e-5)
    out_ref[...] = (z_ref[...] - mean[None, :]) * (rstd * g_ref[0, :])[None, :] \
        + b_ref[0, :][None, :]


def _bn_layer(z, stats, g, b):
    return pl.pallas_call(
        _bn_body,
        grid=(_NBLK,),
        in_specs=[
            pl.BlockSpec((_BLK, HIDDEN), lambda i: (i, 0)),
            pl.BlockSpec((2, HIDDEN), lambda i: (0, 0)),
            pl.BlockSpec((1, HIDDEN), lambda i: (0, 0)),
            pl.BlockSpec((1, HIDDEN), lambda i: (0, 0)),
        ],
        out_specs=pl.BlockSpec((_BLK, HIDDEN), lambda i: (i, 0)),
        out_shape=jax.ShapeDtypeStruct((N_NODES, HIDDEN), jnp.float32),
    )(z, stats, g.reshape(1, HIDDEN), b.reshape(1, HIDDEN))


def _pool_proj_body(batch_ref, m_ref, wp1_ref, bp1_ref, wp2_ref, bp2_ref,
                    out_ref, pool_ref):
    @pl.when(pl.program_id(0) == 0)
    def _():
        pool_ref[...] = jnp.zeros_like(pool_ref)

    gids = lax.broadcasted_iota(jnp.int32, (N_GRAPHS, _BLK), 0)
    onehot = (batch_ref[0, 0, :][None, :] == gids).astype(jnp.float32)
    pool_ref[...] += lax.dot_general(
        onehot, m_ref[...], (((1,), (0,)), ((), ())),
        precision=_PREC_HI, preferred_element_type=jnp.float32)

    @pl.when(pl.program_id(0) == _NBLK - 1)
    def _():
        y = pool_ref[...]
        h1 = jnp.maximum(
            lax.dot_general(y, wp1_ref[...], (((1,), (0,)), ((), ())),
                            precision=_PREC,
                            preferred_element_type=jnp.float32)
            + bp1_ref[...], 0.0)
        out_ref[...] = lax.dot_general(
            h1, wp2_ref[...], (((1,), (0,)), ((), ())),
            precision=_PREC, preferred_element_type=jnp.float32) + bp2_ref[...]


def _pool_proj(batch, m, wp1, bp1, wp2, bp2):
    batch3 = batch.reshape(_NBLK, 1, _BLK)
    return pl.pallas_call(
        _pool_proj_body,
        grid=(_NBLK,),
        in_specs=[
            pl.BlockSpec((1, 1, _BLK), lambda i: (i, 0, 0)),
            pl.BlockSpec((_BLK, EMB), lambda i: (i, 0)),
            pl.BlockSpec((EMB, EMB), lambda i: (0, 0)),
            pl.BlockSpec((1, EMB), lambda i: (0, 0)),
            pl.BlockSpec((EMB, EMB), lambda i: (0, 0)),
            pl.BlockSpec((1, EMB), lambda i: (0, 0)),
        ],
        out_specs=pl.BlockSpec((N_GRAPHS, EMB), lambda i: (0, 0)),
        out_shape=jax.ShapeDtypeStruct((N_GRAPHS, EMB), jnp.float32),
        scratch_shapes=[pltpu.VMEM((N_GRAPHS, EMB), jnp.float32)],
    )(batch3, m, wp1, bp1.reshape(1, EMB), wp2, bp2.reshape(1, EMB))


from jax.experimental.pallas import tpu_sc as plsc

N_EDGES = 160000
_CHUNK = 64
_NCHUNK = N_EDGES // _CHUNK
_RNG = 160
_NRANGE = 64
_NPAD = _RNG * _NRANGE


def _make_sc_agg(d_feat):
    mesh = plsc.VectorSubcoreMesh(core_axis_name="c", subcore_axis_name="s")
    nv = d_feat // 16

    @functools.partial(
        pl.kernel,
        out_type=jax.ShapeDtypeStruct((_NPAD, d_feat), jnp.float32),
        mesh=mesh,
        scratch_types=[
            pltpu.VMEM((_CHUNK,), jnp.int32),
            pltpu.VMEM((_CHUNK,), jnp.int32),
            pltpu.VMEM((_CHUNK, d_feat), jnp.float32),
            pltpu.VMEM((_RNG + 16, d_feat), jnp.float32),
            pltpu.VMEM((16,), jnp.int32),
            pltpu.SemaphoreType.DMA,
        ],
    )
    def sc_agg(h_hbm, srcs_hbm, dsts_hbm, offs_hbm, out_hbm,
               src_i, dst_i, rows, acc, ofs16, sem):
        c = lax.axis_index("c")
        s = lax.axis_index("s")
        wid = c * 16 + s

        pltpu.sync_copy(offs_hbm.at[pl.ds(wid * 16, 16)], ofs16)
        ov = ofs16[pl.ds(0, 16)]

        @pl.loop(0, 2)
        def _pass(p):
            c0 = jnp.where(p == 0, ov[0], ov[2])
            c1 = jnp.where(p == 0, ov[1], ov[3])
            lo = (wid * 2 + p) * _RNG

            @pl.loop(0, _RNG + 16)
            def _za(i):
                for j in range(nv):
                    acc[i, pl.ds(j * 16, 16)] = jnp.zeros((16,), jnp.float32)

            @pl.loop(c0, c1)
            def _chunk(ci):
                pltpu.sync_copy(srcs_hbm.at[pl.ds(ci * _CHUNK, _CHUNK)], src_i)
                pltpu.sync_copy(dsts_hbm.at[pl.ds(ci * _CHUNK, _CHUNK)], dst_i)
                pltpu.async_copy(h_hbm.at[src_i], rows, sem).wait()

                @pl.loop(0, _CHUNK // 16)
                def _grp(g):
                    dv = dst_i[pl.ds(g * 16, 16)]
                    for l in range(16):
                        d0 = dv[l] - lo
                        dl = jnp.where((d0 >= 0) & (d0 < _RNG), d0, _RNG)
                        k = g * 16 + l
                        for j in range(nv):
                            acc[dl, pl.ds(j * 16, 16)] += \
                                rows[k, pl.ds(j * 16, 16)]

            pltpu.sync_copy(acc.at[pl.ds(0, _RNG), :],
                            out_hbm.at[pl.ds((wid * 2 + p) * _RNG, _RNG), :])

    return sc_agg


_sc_agg = {256: _make_sc_agg(256), 512: _make_sc_agg(512)}


def _sort_edges(edge_index):
    order = jnp.argsort(edge_index[1])
    srcs = edge_index[0][order]
    dsts = edge_index[1][order]
    bounds = jnp.searchsorted(
        dsts, jnp.arange(0, _NPAD + 1, _RNG, dtype=jnp.int32)).astype(jnp.int32)
    cs = bounds[:-1] // _CHUNK
    ce = (bounds[1:] + _CHUNK - 1) // _CHUNK
    ofs = jnp.zeros((32, 16), jnp.int32)
    ofs = ofs.at[:, 0].set(cs[0::2]).at[:, 1].set(ce[0::2])
    ofs = ofs.at[:, 2].set(cs[1::2]).at[:, 3].set(ce[1::2])
    return srcs, dsts, ofs.reshape(512)


def _aggregate(h, srcs, dsts, offs):
    return _sc_agg[h.shape[1]](h, srcs, dsts, offs)


def kernel(x, edge_index, batch, params):
    srcs, dsts, offs = _sort_edges(edge_index)
    h = x
    xs = []
    for l in range(N_LAYERS):
        agg = _aggregate(h, srcs, dsts, offs)
        z, stats = _mlp_layer(h, agg, params[f"W1_{l}"], params[f"b1_{l}"],
                              params[f"W2_{l}"], params[f"b2_{l}"])
        z = _bn_layer(z, stats, params[f"bn_g_{l}"], params[f"bn_b_{l}"])
        xs.append(z)
        h = z
    m = jnp.concatenate(xs, axis=1)
    return _pool_proj(batch, m, params["Wp1"], params["bp1"],
                      params["Wp2"], params["bp2"])


# speedup vs baseline: 1.3228x; 1.0003x over previous
import functools

import jax
import jax.numpy as jnp
from jax import lax
from jax.experimental import pallas as pl
from jax.experimental.pallas import tpu as pltpu

N_NODES = 10000
N_GRAPHS = 128
HIDDEN = 512
N_LAYERS = 4
EMB = HIDDEN * N_LAYERS

_BLK = 1000
_NBLK = N_NODES // _BLK
_PREC = lax.Precision.DEFAULT
_PREC_HI = lax.Precision.HIGHEST


def _mlp_body(h_ref, agg_ref, w1_ref, b1_ref, w2_ref, b2_ref,
              z_ref, stats_ref, acc_ref):
    t = h_ref[...] + agg_ref[...]
    z1 = jnp.maximum(
        lax.dot_general(t, w1_ref[...], (((1,), (0,)), ((), ())),
                        precision=_PREC, preferred_element_type=jnp.float32)
        + b1_ref[...], 0.0)
    z2 = lax.dot_general(z1, w2_ref[...], (((1,), (0,)), ((), ())),
                         precision=_PREC, preferred_element_type=jnp.float32) \
        + b2_ref[...]
    z2 = jnp.maximum(z2, 0.0)
    z_ref[...] = z2

    @pl.when(pl.program_id(0) == 0)
    def _():
        acc_ref[...] = jnp.zeros_like(acc_ref)

    acc_ref[0, :] += jnp.sum(z2, axis=0)
    acc_ref[1, :] += jnp.sum(z2 * z2, axis=0)

    @pl.when(pl.program_id(0) == _NBLK - 1)
    def _():
        stats_ref[...] = acc_ref[...]


def _mlp_layer(h, agg, w1, b1, w2, b2):
    d_in = h.shape[1]
    z, stats = pl.pallas_call(
        _mlp_body,
        grid=(_NBLK,),
        in_specs=[
            pl.BlockSpec((_BLK, d_in), lambda i: (i, 0)),
            pl.BlockSpec((_BLK, d_in), lambda i: (i, 0)),
            pl.BlockSpec((d_in, HIDDEN), lambda i: (0, 0)),
            pl.BlockSpec((1, HIDDEN), lambda i: (0, 0)),
            pl.BlockSpec((HIDDEN, HIDDEN), lambda i: (0, 0)),
            pl.BlockSpec((1, HIDDEN), lambda i: (0, 0)),
        ],
        out_specs=[
            pl.BlockSpec((_BLK, HIDDEN), lambda i: (i, 0)),
            pl.BlockSpec((2, HIDDEN), lambda i: (0, 0)),
        ],
        out_shape=[
            jax.ShapeDtypeStruct((N_NODES, HIDDEN), jnp.float32),
            jax.ShapeDtypeStruct((2, HIDDEN), jnp.float32),
        ],
        scratch_shapes=[pltpu.VMEM((2, HIDDEN), jnp.float32)],
    )(h, agg, w1, b1.reshape(1, HIDDEN), w2, b2.reshape(1, HIDDEN))
    return z, stats


def _bn_body(z_ref, stats_ref, g_ref, b_ref, out_ref):
    mean = stats_ref[0, :] / N_NODES
    var = stats_ref[1, :] / N_NODES - mean * mean
    rstd = lax.rsqrt(var + 1e-5)
    out_ref[...] = (z_ref[...] - mean[None, :]) * (rstd * g_ref[0, :])[None, :] \
        + b_ref[0, :][None, :]


def _bn_layer(z, stats, g, b):
    return pl.pallas_call(
        _bn_body,
        grid=(_NBLK,),
        in_specs=[
            pl.BlockSpec((_BLK, HIDDEN), lambda i: (i, 0)),
            pl.BlockSpec((2, HIDDEN), lambda i: (0, 0)),
            pl.BlockSpec((1, HIDDEN), lambda i: (0, 0)),
            pl.BlockSpec((1, HIDDEN), lambda i: (0, 0)),
        ],
        out_specs=pl.BlockSpec((_BLK, HIDDEN), lambda i: (i, 0)),
        out_shape=jax.ShapeDtypeStruct((N_NODES, HIDDEN), jnp.float32),
    )(z, stats, g.reshape(1, HIDDEN), b.reshape(1, HIDDEN))


def _pool_proj_body(batch_ref, m_ref, wp1_ref, bp1_ref, wp2_ref, bp2_ref,
                    out_ref, pool_ref):
    @pl.when(pl.program_id(0) == 0)
    def _():
        pool_ref[...] = jnp.zeros_like(pool_ref)

    gids = lax.broadcasted_iota(jnp.int32, (N_GRAPHS, _BLK), 0)
    onehot = (batch_ref[0, 0, :][None, :] == gids).astype(jnp.float32)
    pool_ref[...] += lax.dot_general(
        onehot, m_ref[...], (((1,), (0,)), ((), ())),
        precision=_PREC_HI, preferred_element_type=jnp.float32)

    @pl.when(pl.program_id(0) == _NBLK - 1)
    def _():
        y = pool_ref[...]
        h1 = jnp.maximum(
            lax.dot_general(y, wp1_ref[...], (((1,), (0,)), ((), ())),
                            precision=_PREC,
                            preferred_element_type=jnp.float32)
            + bp1_ref[...], 0.0)
        out_ref[...] = lax.dot_general(
            h1, wp2_ref[...], (((1,), (0,)), ((), ())),
            precision=_PREC, preferred_element_type=jnp.float32) + bp2_ref[...]


def _pool_proj(batch, m, wp1, bp1, wp2, bp2):
    batch3 = batch.reshape(_NBLK, 1, _BLK)
    return pl.pallas_call(
        _pool_proj_body,
        grid=(_NBLK,),
        in_specs=[
            pl.BlockSpec((1, 1, _BLK), lambda i: (i, 0, 0)),
            pl.BlockSpec((_BLK, EMB), lambda i: (i, 0)),
            pl.BlockSpec((EMB, EMB), lambda i: (0, 0)),
            pl.BlockSpec((1, EMB), lambda i: (0, 0)),
            pl.BlockSpec((EMB, EMB), lambda i: (0, 0)),
            pl.BlockSpec((1, EMB), lambda i: (0, 0)),
        ],
        out_specs=pl.BlockSpec((N_GRAPHS, EMB), lambda i: (0, 0)),
        out_shape=jax.ShapeDtypeStruct((N_GRAPHS, EMB), jnp.float32),
        scratch_shapes=[pltpu.VMEM((N_GRAPHS, EMB), jnp.float32)],
    )(batch3, m, wp1, bp1.reshape(1, EMB), wp2, bp2.reshape(1, EMB))


from jax.experimental.pallas import tpu_sc as plsc

N_EDGES = 160000
_CHUNK = 64
_NCHUNK = N_EDGES // _CHUNK
_RNG = 160
_NRANGE = 64
_NPAD = _RNG * _NRANGE


def _make_sc_agg(d_feat):
    mesh = plsc.VectorSubcoreMesh(core_axis_name="c", subcore_axis_name="s")
    nv = d_feat // 16

    @functools.partial(
        pl.kernel,
        out_type=jax.ShapeDtypeStruct((_NPAD, d_feat), jnp.float32),
        mesh=mesh,
        scratch_types=[
            pltpu.VMEM((_CHUNK,), jnp.int32),
            pltpu.VMEM((_CHUNK,), jnp.int32),
            pltpu.VMEM((_CHUNK, d_feat), jnp.float32),
            pltpu.VMEM((_RNG + 16, d_feat), jnp.float32),
            pltpu.VMEM((16,), jnp.int32),
            pltpu.SemaphoreType.DMA,
        ],
    )
    def sc_agg(h_hbm, srcs_hbm, dsts_hbm, offs_hbm, out_hbm,
               src_i, dst_i, rows, acc, ofs16, sem):
        c = lax.axis_index("c")
        s = lax.axis_index("s")
        wid = c * 16 + s

        pltpu.sync_copy(offs_hbm.at[pl.ds(wid * 16, 16)], ofs16)
        ov = ofs16[pl.ds(0, 16)]

        @pl.loop(0, 2)
        def _pass(p):
            c0 = jnp.where(p == 0, ov[0], ov[2])
            c1 = jnp.where(p == 0, ov[1], ov[3])
            lo = (wid * 2 + p) * _RNG

            @pl.loop(0, _RNG + 16)
            def _za(i):
                for j in range(nv):
                    acc[i, pl.ds(j * 16, 16)] = jnp.zeros((16,), jnp.float32)

            @pl.loop(c0, c1)
            def _chunk(ci):
                pltpu.sync_copy(srcs_hbm.at[pl.ds(ci * _CHUNK, _CHUNK)], src_i)
                pltpu.sync_copy(dsts_hbm.at[pl.ds(ci * _CHUNK, _CHUNK)], dst_i)
                pltpu.async_copy(h_hbm.at[src_i], rows, sem).wait()

                @pl.loop(0, _CHUNK // 16)
                def _grp(g):
                    dv = dst_i[pl.ds(g * 16, 16)]
                    for l in range(16):
                        d0 = dv[l] - lo
                        dl = jnp.where((d0 >= 0) & (d0 < _RNG), d0, _RNG)
                        k = g * 16 + l
                        arow = acc.at[dl]
                        rrow = rows.at[k]
                        for j in range(nv):
                            arow[pl.ds(j * 16, 16)] += rrow[pl.ds(j * 16, 16)]

            pltpu.sync_copy(acc.at[pl.ds(0, _RNG), :],
                            out_hbm.at[pl.ds((wid * 2 + p) * _RNG, _RNG), :])

    return sc_agg


_sc_agg = {256: _make_sc_agg(256), 512: _make_sc_agg(512)}


def _sort_edges(edge_index):
    order = jnp.argsort(edge_index[1])
    srcs = edge_index[0][order]
    dsts = edge_index[1][order]
    bounds = jnp.searchsorted(
        dsts, jnp.arange(0, _NPAD + 1, _RNG, dtype=jnp.int32)).astype(jnp.int32)
    cs = bounds[:-1] // _CHUNK
    ce = (bounds[1:] + _CHUNK - 1) // _CHUNK
    ofs = jnp.zeros((32, 16), jnp.int32)
    ofs = ofs.at[:, 0].set(cs[0::2]).at[:, 1].set(ce[0::2])
    ofs = ofs.at[:, 2].set(cs[1::2]).at[:, 3].set(ce[1::2])
    return srcs, dsts, ofs.reshape(512)


def _aggregate(h, srcs, dsts, offs):
    return _sc_agg[h.shape[1]](h, srcs, dsts, offs)


def kernel(x, edge_index, batch, params):
    srcs, dsts, offs = _sort_edges(edge_index)
    h = x
    xs = []
    for l in range(N_LAYERS):
        agg = _aggregate(h, srcs, dsts, offs)
        z, stats = _mlp_layer(h, agg, params[f"W1_{l}"], params[f"b1_{l}"],
                              params[f"W2_{l}"], params[f"b2_{l}"])
        z = _bn_layer(z, stats, params[f"bn_g_{l}"], params[f"bn_b_{l}"])
        xs.append(z)
        h = z
    m = jnp.concatenate(xs, axis=1)
    return _pool_proj(batch, m, params["Wp1"], params["bp1"],
                      params["Wp2"], params["bp2"])
